# Initial kernel scaffold; baseline (speedup 1.0000x reference)
import jax
import jax.numpy as jnp
from jax.experimental import pallas as pl


def kernel(hidden_states, weight):
    raise NotImplementedError("write your pallas kernel here")


# speedup vs baseline: 1.2587x; 1.2587x over previous
import functools

import jax
import jax.numpy as jnp
from jax.experimental import pallas as pl
from jax.experimental.pallas import tpu as pltpu

EMBED_DIM = 768
N_EXPERTS = 64
TOP_K = 8
ALPHA = 0.01


def _router_block(x_ref, wt_ref, idx_ref, w_ref, aux_ref, ssum, scnt):
    step = pl.program_id(0)
    nsteps = pl.num_programs(0)

    x = x_ref[...]
    wt = wt_ref[...]
    logits = jax.lax.dot_general(
        x, wt, (((1,), (0,)), ((), ())), preferred_element_type=jnp.float32
    )

    m = jnp.max(logits, axis=-1, keepdims=True)
    e = jnp.exp(logits - m)
    scores = e / jnp.sum(e, axis=-1, keepdims=True)

    bt = scores.shape[0]
    iota = jax.lax.broadcasted_iota(jnp.int32, (bt, N_EXPERTS), 1)
    work = scores
    sel = jnp.zeros((bt, N_EXPERTS), jnp.float32)
    for k in range(TOP_K):
        mk = jnp.max(work, axis=-1, keepdims=True)
        hit = work >= mk
        ik = jnp.min(jnp.where(hit, iota, N_EXPERTS), axis=-1, keepdims=True)
        onehot = (iota == ik).astype(jnp.float32)
        sel = sel + onehot
        idx_ref[:, k] = ik[:, 0]
        w_ref[:, k] = mk[:, 0]
        work = work - onehot * work

    @pl.when(step == 0)
    def _init():
        ssum[...] = jnp.zeros_like(ssum)
        scnt[...] = jnp.zeros_like(scnt)

    ssum[...] += jnp.sum(scores, axis=0, keepdims=True)
    scnt[...] += jnp.sum(sel, axis=0, keepdims=True)

    @pl.when(step == nsteps - 1)
    def _fin():
        t = jnp.float32(nsteps * bt)
        scale = ALPHA * N_EXPERTS / (t * t * TOP_K)
        aux_ref[0, 0] = jnp.sum(ssum[...] * scnt[...]) * scale


@jax.jit
def kernel(hidden_states, weight):
    bsz, seq_len, h = hidden_states.shape
    tokens = bsz * seq_len
    x = hidden_states.reshape(tokens, h)
    wt = weight.T

    bt = 1024
    grid = (tokens // bt,)

    idx, w, aux = pl.pallas_call(
        _router_block,
        grid=grid,
        in_specs=[
            pl.BlockSpec((bt, h), lambda i: (i, 0)),
            pl.BlockSpec((h, N_EXPERTS), lambda i: (0, 0)),
        ],
        out_specs=[
            pl.BlockSpec((bt, TOP_K), lambda i: (i, 0)),
            pl.BlockSpec((bt, TOP_K), lambda i: (i, 0)),
            pl.BlockSpec(memory_space=pltpu.SMEM),
        ],
        out_shape=[
            jax.ShapeDtypeStruct((tokens, TOP_K), jnp.int32),
            jax.ShapeDtypeStruct((tokens, TOP_K), jnp.float32),
            jax.ShapeDtypeStruct((1, 1), jnp.float32),
        ],
        scratch_shapes=[
            pltpu.VMEM((1, N_EXPERTS), jnp.float32),
            pltpu.VMEM((1, N_EXPERTS), jnp.float32),
        ],
    )(x, wt)
    return idx, w, aux[0, 0]


# speedup vs baseline: 2.5771x; 2.0473x over previous
import jax
import jax.numpy as jnp
from jax.experimental import pallas as pl
from jax.experimental.pallas import tpu as pltpu

EMBED_DIM = 768
N_EXPERTS = 64
TOP_K = 8
ALPHA = 0.01


def _router_block(x_ref, w_ref, idx_ref, wout_ref, aux_ref, ssum, scnt):
    step = pl.program_id(0)
    nsteps = pl.num_programs(0)

    x = x_ref[...]
    w = w_ref[...]
    logits_t = jax.lax.dot_general(
        w, x, (((1,), (1,)), ((), ())), preferred_element_type=jnp.float32
    )

    m = jnp.max(logits_t, axis=0, keepdims=True)
    e = jnp.exp(logits_t - m)
    s = jnp.sum(e, axis=0, keepdims=True)
    scores = e / s

    bt = scores.shape[1]
    eidx = jax.lax.broadcasted_iota(jnp.int32, (N_EXPERTS, bt), 0)
    keys = (jax.lax.bitcast_convert_type(scores, jnp.int32) & -64) | (63 - eidx)

    kiota = jax.lax.broadcasted_iota(jnp.int32, (TOP_K, bt), 0)
    idx_t = jnp.zeros((TOP_K, bt), jnp.int32)
    w_t = jnp.zeros((TOP_K, bt), jnp.float32)
    for k in range(TOP_K):
        mk = jnp.max(keys, axis=0, keepdims=True)
        keys = jnp.where(keys == mk, -1, keys)
        ik = 63 - (mk & 63)
        wk = jax.lax.bitcast_convert_type(mk & -64, jnp.float32)
        idx_t = jnp.where(kiota == k, ik, idx_t)
        w_t = jnp.where(kiota == k, wk, w_t)

    idx_ref[...] = idx_t.T
    wout_ref[...] = w_t.T

    sel = (keys < 0).astype(jnp.float32)

    @pl.when(step == 0)
    def _init():
        ssum[...] = jnp.zeros_like(ssum)
        scnt[...] = jnp.zeros_like(scnt)

    ssum[...] += jnp.sum(scores, axis=1, keepdims=True)
    scnt[...] += jnp.sum(sel, axis=1, keepdims=True)

    @pl.when(step == nsteps - 1)
    def _fin():
        t = jnp.float32(nsteps * bt)
        scale = ALPHA * N_EXPERTS / (t * t * TOP_K)
        aux_ref[0, 0] = jnp.sum(ssum[...] * scnt[...]) * scale


@jax.jit
def kernel(hidden_states, weight):
    bsz, seq_len, h = hidden_states.shape
    tokens = bsz * seq_len
    x = hidden_states.reshape(tokens, h)

    bt = 1024
    grid = (tokens // bt,)

    idx, wout, aux = pl.pallas_call(
        _router_block,
        grid=grid,
        in_specs=[
            pl.BlockSpec((bt, h), lambda i: (i, 0)),
            pl.BlockSpec((N_EXPERTS, h), lambda i: (0, 0)),
        ],
        out_specs=[
            pl.BlockSpec((bt, TOP_K), lambda i: (i, 0)),
            pl.BlockSpec((bt, TOP_K), lambda i: (i, 0)),
            pl.BlockSpec(memory_space=pltpu.SMEM),
        ],
        out_shape=[
            jax.ShapeDtypeStruct((tokens, TOP_K), jnp.int32),
            jax.ShapeDtypeStruct((tokens, TOP_K), jnp.float32),
            jax.ShapeDtypeStruct((1, 1), jnp.float32),
        ],
        scratch_shapes=[
            pltpu.VMEM((N_EXPERTS, 1), jnp.float32),
            pltpu.VMEM((N_EXPERTS, 1), jnp.float32),
        ],
    )(x, weight)
    return idx, wout, aux[0, 0]
